# Initial kernel scaffold; baseline (speedup 1.0000x reference)
import jax
import jax.numpy as jnp
from jax.experimental import pallas as pl


def kernel(indices, tables):
    raise NotImplementedError("write your pallas kernel here")


# speedup vs baseline: 1.2225x; 1.2225x over previous
import functools

import jax
import jax.numpy as jnp
from jax import lax
from jax.experimental import pallas as pl
from jax.experimental.pallas import tpu as pltpu
from jax.experimental.pallas import tpu_sc as plsc

LANES = 16
IDX_CHUNK = 128


def kernel(indices, tables):
    T, B, G = indices.shape
    V, D = tables.shape[1], tables.shape[2]
    info = plsc.get_sparse_core_info()
    NC, NS = info.num_cores, info.num_subcores
    NW = NC * NS
    bpw = B // NW
    rpw = bpw * G
    nch = rpw // IDX_CHUNK
    dh = D // LANES

    flat_tables = tables.reshape(T * V, D)
    idx_w = (
        indices.reshape(T, NW, rpw)
        .transpose(1, 0, 2)
        .reshape(NW, T * nch, IDX_CHUNK)
    )

    mesh = plsc.VectorSubcoreMesh(core_axis_name="c", subcore_axis_name="s")

    @functools.partial(
        pl.kernel,
        mesh=mesh,
        compiler_params=pltpu.CompilerParams(use_tc_tiling_on_sc=False),
        out_type=jax.ShapeDtypeStruct((B, T * D), jnp.float32),
        scratch_types=[
            pltpu.VMEM((T * nch, IDX_CHUNK), jnp.int32),
            pltpu.VMEM((rpw, D), jnp.float32),
            pltpu.VMEM((bpw, T * D), jnp.float32),
            pltpu.SemaphoreType.DMA,
        ],
    )
    def ebag(idx_hbm, tab_hbm, out_hbm, idx_v, rows_v, out_v, sem):
        wid = lax.axis_index("s") * NC + lax.axis_index("c")
        pltpu.sync_copy(idx_hbm.at[wid], idx_v)

        def table_body(t, carry):
            off = t * V
            for g in range(nch):
                for c in range(IDX_CHUNK // LANES):
                    sl = pl.ds(c * LANES, LANES)
                    idx_v[t * nch + g, sl] = idx_v[t * nch + g, sl] + off
            cps = [
                pltpu.async_copy(
                    tab_hbm.at[idx_v.at[t * nch + g]],
                    rows_v.at[pl.ds(g * IDX_CHUNK, IDX_CHUNK)],
                    sem,
                )
                for g in range(nch)
            ]
            for cp in cps:
                cp.wait()

            def sample_body(s, c2):
                base = s * G
                for h in range(dh):
                    sl = pl.ds(h * LANES, LANES)
                    acc = rows_v[base, sl]
                    for j in range(1, G):
                        acc = acc + rows_v[base + j, sl]
                    out_v[s, pl.ds(t * D + h * LANES, LANES)] = acc
                return c2

            lax.fori_loop(0, bpw, sample_body, 0)
            return carry

        lax.fori_loop(0, T, table_body, 0)
        pltpu.sync_copy(out_v, out_hbm.at[pl.ds(wid * bpw, bpw)])

    return ebag(idx_w, flat_tables)


# speedup vs baseline: 1.2564x; 1.0278x over previous
import functools

import jax
import jax.numpy as jnp
from jax import lax
from jax.experimental import pallas as pl
from jax.experimental.pallas import tpu as pltpu
from jax.experimental.pallas import tpu_sc as plsc

LANES = 16
IDX_CHUNK = 128


def kernel(indices, tables):
    T, B, G = indices.shape
    V, D = tables.shape[1], tables.shape[2]
    info = plsc.get_sparse_core_info()
    NC, NS = info.num_cores, info.num_subcores
    NW = NC * NS
    bpw = B // NW
    rpw = bpw * G
    nch = rpw // IDX_CHUNK
    dh = D // LANES

    flat_tables = tables.reshape(T * V, D)
    idx_w = (
        indices.reshape(T, NW, rpw)
        .transpose(1, 0, 2)
        .reshape(NW, T * nch, IDX_CHUNK)
    )

    mesh = plsc.VectorSubcoreMesh(core_axis_name="c", subcore_axis_name="s")

    @functools.partial(
        pl.kernel,
        mesh=mesh,
        compiler_params=pltpu.CompilerParams(use_tc_tiling_on_sc=False),
        out_type=jax.ShapeDtypeStruct((B, T * D), jnp.float32),
        scratch_types=[
            pltpu.VMEM((T * nch, IDX_CHUNK), jnp.int32),
            pltpu.VMEM((2 * rpw, D), jnp.float32),
            pltpu.VMEM((bpw, T * D), jnp.float32),
            pltpu.SemaphoreType.DMA,
            pltpu.SemaphoreType.DMA,
        ],
    )
    def ebag(idx_hbm, tab_hbm, out_hbm, idx_v, rows_v, out_v, sem0, sem1):
        wid = lax.axis_index("s") * NC + lax.axis_index("c")
        sems = (sem0, sem1)
        pltpu.sync_copy(idx_hbm.at[wid], idx_v)

        def add_off(t):
            off = t * V
            for g in range(nch):
                row = t * nch + g
                for c in range(IDX_CHUNK // LANES):
                    sl = pl.ds(c * LANES, LANES)
                    idx_v[row, sl] = idx_v[row, sl] + off

        def copies(t, buf):
            return [
                pltpu.make_async_copy(
                    tab_hbm.at[idx_v.at[t * nch + g]],
                    rows_v.at[pl.ds(buf * rpw + g * IDX_CHUNK, IDX_CHUNK)],
                    sems[buf],
                )
                for g in range(nch)
            ]

        def issue(t, buf):
            add_off(t)
            for cp in copies(t, buf):
                cp.start()

        def drain(t, buf):
            for cp in copies(t, buf):
                cp.wait()

        def pool(t, buf):
            def sample_body(s, c2):
                base = buf * rpw + s * G
                for h in range(dh):
                    sl = pl.ds(h * LANES, LANES)
                    acc = rows_v[base, sl]
                    for j in range(1, G):
                        acc = acc + rows_v[base + j, sl]
                    out_v[s, pl.ds(t * D + h * LANES, LANES)] = acc
                return c2

            lax.fori_loop(0, bpw, sample_body, 0)

        issue(0, 0)

        def pair_body(i, carry):
            t0 = 2 * i
            t1 = t0 + 1
            issue(t1, 1)
            drain(t0, 0)
            pool(t0, 0)

            @pl.when(t0 + 2 < T)
            def _():
                issue(t0 + 2, 0)

            drain(t1, 1)
            pool(t1, 1)
            return carry

        lax.fori_loop(0, T // 2, pair_body, 0)
        pltpu.sync_copy(out_v, out_hbm.at[pl.ds(wid * bpw, bpw)])

    return ebag(idx_w, flat_tables)
